# Initial kernel scaffold; baseline (speedup 1.0000x reference)
import jax
import jax.numpy as jnp
from jax.experimental import pallas as pl


def kernel(X, table):
    raise NotImplementedError("write your pallas kernel here")


# speedup vs baseline: 1.8773x; 1.8773x over previous
import functools

import jax
import jax.numpy as jnp
from jax import lax
from jax.experimental import pallas as pl
from jax.experimental.pallas import tpu as pltpu
from jax.experimental.pallas import tpu_sc as plsc

NC = 2
NS = 16
NW = NC * NS
CH = 128
NB = 4


@functools.lru_cache(maxsize=None)
def _build(N, D):
    assert N % (NW * CH) == 0
    b_per_w = N // NW
    nch = b_per_w // CH
    ngroups = nch // NB
    assert nch % NB == 0
    mesh = plsc.VectorSubcoreMesh(core_axis_name="c", subcore_axis_name="s")

    @functools.partial(
        pl.kernel,
        out_type=jax.ShapeDtypeStruct((N, D), jnp.float32),
        mesh=mesh,
        compiler_params=pltpu.CompilerParams(use_tc_tiling_on_sc=False),
        scratch_types=[
            pltpu.VMEM((nch, CH), jnp.int32),
            pltpu.VMEM((NB, CH, D), jnp.float32),
        ] + [pltpu.SemaphoreType.DMA] * NB,
    )
    def emb(idx_hbm, table_hbm, out_hbm, idx_v, rows_v, *gsems):
        wid = lax.axis_index("s") * NC + lax.axis_index("c")
        base = wid * b_per_w
        pltpu.sync_copy(idx_hbm.at[pl.ds(wid * nch, nch)], idx_v)

        for b in range(NB):
            pltpu.async_copy(table_hbm.at[idx_v.at[b]], rows_v.at[b], gsems[b])

        def group(p, carry):
            for b in range(NB):
                j = p * NB + b
                pltpu.make_async_copy(
                    table_hbm.at[idx_v.at[j]], rows_v.at[b], gsems[b]
                ).wait()
                pltpu.sync_copy(
                    rows_v.at[b], out_hbm.at[pl.ds(base + j * CH, CH)]
                )
                jn = j + NB

                @pl.when(jn < nch)
                def _():
                    pltpu.async_copy(
                        table_hbm.at[idx_v.at[jn]], rows_v.at[b], gsems[b]
                    )
            return carry

        lax.fori_loop(0, ngroups, group, 0)

    return emb


def kernel(X, table):
    N = X.size
    D = table.shape[1]
    idx2d = X.reshape(N // CH, CH).astype(jnp.int32)
    out = _build(N, D)(idx2d, table)
    return out.reshape(*X.shape, D)
